# Initial kernel scaffold; baseline (speedup 1.0000x reference)
import jax
import jax.numpy as jnp
from jax.experimental import pallas as pl


def kernel(input_tensor):
    raise NotImplementedError("write your pallas kernel here")


# speedup vs baseline: 4.1169x; 4.1169x over previous
import functools

import jax
import jax.numpy as jnp
from jax import lax
from jax.experimental import pallas as pl
from jax.experimental.pallas import tpu as pltpu
from jax.experimental.pallas import tpu_sc as plsc

B = 32
N = 8192
KSEL = 256
L = 16
NCHUNK = N // L
NC = 2
NS = 16

HI_BASE = 0
LO_BASE = N + 32
BUFLEN = LO_BASE + N + 32

BITS_HI = 0x7F800001


def _partition_pass(src, dst, state, lanes):
    lo, hi, goal, off, n = state
    done = (hi - lo) <= 1
    mid = lo + ((hi - lo) >> 1)
    nch = jnp.where(done, 0, (n + L - 1) // L)

    def body(i, carry):
        phi, plo = carry
        base = i * L
        bits = src[pl.ds(off + base, L)]
        valid = lanes < (n - base)
        mhi = valid & (bits >= mid)
        mlo = valid & (bits < mid)
        chi = jnp.sum(mhi.astype(jnp.int32))
        clo = jnp.sum(mlo.astype(jnp.int32))
        plsc.store_compressed(dst.at[pl.ds(HI_BASE + phi, L)], bits, mask=mhi)
        plsc.store_compressed(dst.at[pl.ds(LO_BASE + plo, L)], bits, mask=mlo)
        return (phi + chi, plo + clo)

    chi_t, clo_t = lax.fori_loop(0, nch, body, (jnp.int32(0), jnp.int32(0)))

    take_hi = chi_t >= goal
    lo2 = jnp.where(done, lo, jnp.where(take_hi, mid, lo))
    hi2 = jnp.where(done, hi, jnp.where(take_hi, hi, mid))
    goal2 = jnp.where(done | take_hi, goal, goal - chi_t)
    off2 = jnp.where(done, off, jnp.where(take_hi, HI_BASE, LO_BASE))
    n2 = jnp.where(done, n, jnp.where(take_hi, chi_t, clo_t))
    return (lo2, hi2, goal2, off2, n2)


def _kwta_body(x_hbm, out_hbm, row_v, b0, b1):
    wid = lax.axis_index("s") * NC + lax.axis_index("c")
    pltpu.sync_copy(x_hbm.at[wid], row_v)

    lanes = lax.iota(jnp.int32, L)

    def init_body(i, _):
        xc = row_v[pl.ds(i * L, L)]
        b0[pl.ds(i * L, L)] = plsc.bitcast(xc, jnp.int32) & jnp.int32(0x7FFFFFFF)
        return 0

    lax.fori_loop(0, NCHUNK, init_body, 0)

    state = (jnp.int32(0), jnp.int32(BITS_HI), jnp.int32(KSEL),
             jnp.int32(HI_BASE), jnp.int32(N))

    def dbl(i, st):
        st = _partition_pass(b0, b1, st, lanes)
        st = _partition_pass(b1, b0, st, lanes)
        return st

    state = lax.fori_loop(0, 16, dbl, state)
    thresh = state[0]

    def fin_body(i, _):
        xc = row_v[pl.ds(i * L, L)]
        bits = plsc.bitcast(xc, jnp.int32) & jnp.int32(0x7FFFFFFF)
        row_v[pl.ds(i * L, L)] = jnp.where(bits >= thresh, xc, jnp.float32(0.0))
        return 0

    lax.fori_loop(0, NCHUNK, fin_body, 0)
    pltpu.sync_copy(row_v, out_hbm.at[wid])


@functools.cache
def _kwta():
    mesh = plsc.VectorSubcoreMesh(core_axis_name="c", subcore_axis_name="s",
                                  num_cores=NC, num_subcores=NS)
    return pl.kernel(
        _kwta_body,
        out_type=jax.ShapeDtypeStruct((B, N), jnp.float32),
        mesh=mesh,
        scratch_types=[
            pltpu.VMEM((N,), jnp.float32),
            pltpu.VMEM((BUFLEN,), jnp.int32),
            pltpu.VMEM((BUFLEN,), jnp.int32),
        ],
        compiler_params=pltpu.CompilerParams(needs_layout_passes=False),
    )


@jax.jit
def kernel(input_tensor):
    return _kwta()(input_tensor)


# speedup vs baseline: 5.1136x; 1.2421x over previous
import functools

import jax
import jax.numpy as jnp
from jax import lax
from jax.experimental import pallas as pl
from jax.experimental.pallas import tpu as pltpu
from jax.experimental.pallas import tpu_sc as plsc

B = 32
N = 8192
KSEL = 256
L = 16
NCHUNK = N // L
NC = 2
NS = 16

HI_BASE = 0
LO_BASE = N + 32
BUFLEN = LO_BASE + N + 32

BITS_HI = 0x7F800001


def _partition_pass(src, dst, state, lanes):
    lo, hi, goal, off, n = state
    done = (hi - lo) <= 1
    mid = lo + ((hi - lo) >> 1)
    nch = jnp.where(done, 0, (n + L - 1) // L)

    def body(i, carry):
        phi, plo = carry
        base = i * L
        bits = src[pl.ds(off + base, L)]
        valid = lanes < (n - base)
        mhi = valid & (bits >= mid)
        mlo = valid & (bits < mid)
        chi = jnp.sum(mhi.astype(jnp.int32))
        clo = jnp.sum(mlo.astype(jnp.int32))
        plsc.store_compressed(dst.at[pl.ds(HI_BASE + phi, L)], bits, mask=mhi)
        plsc.store_compressed(dst.at[pl.ds(LO_BASE + plo, L)], bits, mask=mlo)
        return (phi + chi, plo + clo)

    chi_t, clo_t = lax.fori_loop(0, nch, body, (jnp.int32(0), jnp.int32(0)))

    take_hi = chi_t >= goal
    lo2 = jnp.where(done, lo, jnp.where(take_hi, mid, lo))
    hi2 = jnp.where(done, hi, jnp.where(take_hi, hi, mid))
    goal2 = jnp.where(done | take_hi, goal, goal - chi_t)
    off2 = jnp.where(done, off, jnp.where(take_hi, HI_BASE, LO_BASE))
    n2 = jnp.where(done, n, jnp.where(take_hi, chi_t, clo_t))
    return (lo2, hi2, goal2, off2, n2)


def _kwta_body(x_hbm, out_hbm, row_v, b0, b1):
    wid = lax.axis_index("s") * NC + lax.axis_index("c")
    pltpu.sync_copy(x_hbm.at[wid], row_v)

    lanes = lax.iota(jnp.int32, L)

    MID1 = BITS_HI >> 1

    def p1_body(i, carry):
        phi, plo, mx = carry
        xc = row_v[pl.ds(i * L, L)]
        bits = plsc.bitcast(xc, jnp.int32) & jnp.int32(0x7FFFFFFF)
        mhi = bits >= MID1
        chi = jnp.sum(mhi.astype(jnp.int32))
        plsc.store_compressed(b0.at[pl.ds(HI_BASE + phi, L)], bits, mask=mhi)
        plsc.store_compressed(b0.at[pl.ds(LO_BASE + plo, L)], bits,
                              mask=jnp.logical_not(mhi))
        return (phi + chi, plo + (L - chi), jnp.maximum(mx, bits))

    phi, plo, mxv = lax.fori_loop(
        0, NCHUNK, p1_body,
        (jnp.int32(0), jnp.int32(0), jnp.zeros((L,), jnp.int32)))
    hi_cap = jnp.max(mxv) + 1

    take_hi = phi >= KSEL
    state = (
        jnp.where(take_hi, jnp.int32(MID1), jnp.int32(0)),
        jnp.minimum(jnp.where(take_hi, jnp.int32(BITS_HI), jnp.int32(MID1)),
                    hi_cap),
        jnp.where(take_hi, jnp.int32(KSEL), KSEL - phi),
        jnp.where(take_hi, jnp.int32(HI_BASE), jnp.int32(LO_BASE)),
        jnp.where(take_hi, phi, plo),
    )

    def dbl(i, st):
        st = _partition_pass(b0, b1, st, lanes)
        st = _partition_pass(b1, b0, st, lanes)
        return st

    state = lax.fori_loop(0, 16, dbl, state)
    thresh = state[0]

    def fin_body(i, _):
        xc = row_v[pl.ds(i * L, L)]
        bits = plsc.bitcast(xc, jnp.int32) & jnp.int32(0x7FFFFFFF)
        row_v[pl.ds(i * L, L)] = jnp.where(bits >= thresh, xc, jnp.float32(0.0))
        return 0

    lax.fori_loop(0, NCHUNK, fin_body, 0)
    pltpu.sync_copy(row_v, out_hbm.at[wid])


@functools.cache
def _kwta():
    mesh = plsc.VectorSubcoreMesh(core_axis_name="c", subcore_axis_name="s",
                                  num_cores=NC, num_subcores=NS)
    return pl.kernel(
        _kwta_body,
        out_type=jax.ShapeDtypeStruct((B, N), jnp.float32),
        mesh=mesh,
        scratch_types=[
            pltpu.VMEM((N,), jnp.float32),
            pltpu.VMEM((BUFLEN,), jnp.int32),
            pltpu.VMEM((BUFLEN,), jnp.int32),
        ],
        compiler_params=pltpu.CompilerParams(needs_layout_passes=False),
    )


@jax.jit
def kernel(input_tensor):
    return _kwta()(input_tensor)


# speedup vs baseline: 6.3403x; 1.2399x over previous
import functools

import jax
import jax.numpy as jnp
from jax import lax
from jax.experimental import pallas as pl
from jax.experimental.pallas import tpu as pltpu
from jax.experimental.pallas import tpu_sc as plsc

B = 32
N = 8192
KSEL = 256
L = 16
NCHUNK = N // L
NC = 2
NS = 16

SEG = N + 32
BUF4LEN = 4 * SEG
BUF2LEN = 2 * SEG
HI_BASE = 0
LO_BASE = SEG

BITS_HI = 0x7F800001

P1 = 0x3FF00000
P2 = 0x40080000
P3 = 0x4019999A


def _partition_pass(src, dst, state, lanes):
    lo, hi, goal, off, n = state
    done = (hi - lo) <= 1
    mid = lo + ((hi - lo) >> 1)
    nch = jnp.where(done, 0, (n + L - 1) // L)

    def body(i, carry):
        phi, plo = carry
        base = i * L
        bits = src[pl.ds(off + base, L)]
        valid = lanes < (n - base)
        mhi = valid & (bits >= mid)
        mlo = valid & (bits < mid)
        chi = jnp.sum(mhi.astype(jnp.int32))
        clo = jnp.sum(mlo.astype(jnp.int32))
        plsc.store_compressed(dst.at[pl.ds(HI_BASE + phi, L)], bits, mask=mhi)
        plsc.store_compressed(dst.at[pl.ds(LO_BASE + plo, L)], bits, mask=mlo)
        return (phi + chi, plo + clo)

    chi_t, clo_t = lax.fori_loop(0, nch, body, (jnp.int32(0), jnp.int32(0)))

    take_hi = chi_t >= goal
    lo2 = jnp.where(done, lo, jnp.where(take_hi, mid, lo))
    hi2 = jnp.where(done, hi, jnp.where(take_hi, hi, mid))
    goal2 = jnp.where(done | take_hi, goal, goal - chi_t)
    off2 = jnp.where(done, off, jnp.where(take_hi, HI_BASE, LO_BASE))
    n2 = jnp.where(done, n, jnp.where(take_hi, chi_t, clo_t))
    return (lo2, hi2, goal2, off2, n2)


def _kwta_body(x_hbm, out_hbm, row_v, b0, b1):
    wid = lax.axis_index("s") * NC + lax.axis_index("c")
    pltpu.sync_copy(x_hbm.at[wid], row_v)

    lanes = lax.iota(jnp.int32, L)

    def p1_body(i, carry):
        q0, q1, q2, q3, mx = carry
        xc = row_v[pl.ds(i * L, L)]
        bits = plsc.bitcast(xc, jnp.int32) & jnp.int32(0x7FFFFFFF)
        ge1 = bits >= P1
        ge2 = bits >= P2
        ge3 = bits >= P3
        m3 = ge3
        m2 = ge2 & jnp.logical_not(ge3)
        m1 = ge1 & jnp.logical_not(ge2)
        m0 = jnp.logical_not(ge1)
        c1 = jnp.sum(m1.astype(jnp.int32))
        c2 = jnp.sum(m2.astype(jnp.int32))
        c3 = jnp.sum(m3.astype(jnp.int32))
        c0 = L - c1 - c2 - c3
        plsc.store_compressed(b0.at[pl.ds(0 * SEG + q0, L)], bits, mask=m0)
        plsc.store_compressed(b0.at[pl.ds(1 * SEG + q1, L)], bits, mask=m1)
        plsc.store_compressed(b0.at[pl.ds(2 * SEG + q2, L)], bits, mask=m2)
        plsc.store_compressed(b0.at[pl.ds(3 * SEG + q3, L)], bits, mask=m3)
        return (q0 + c0, q1 + c1, q2 + c2, q3 + c3, jnp.maximum(mx, bits))

    z = jnp.int32(0)
    q0, q1, q2, q3, mxv = plsc.parallel_loop(
        0, NCHUNK, carry=(z, z, z, z, jnp.zeros((L,), jnp.int32)),
        unroll=2)(p1_body)

    hi_cap = jnp.max(mxv) + 1
    c_ge_p3 = q3
    c_ge_p2 = q3 + q2
    c_ge_p1 = c_ge_p2 + q1
    in3 = c_ge_p3 >= KSEL
    in2 = jnp.logical_not(in3) & (c_ge_p2 >= KSEL)
    in1 = jnp.logical_not(in3) & jnp.logical_not(in2) & (c_ge_p1 >= KSEL)
    sel3 = lambda a, b, c, d: jnp.where(
        in3, a, jnp.where(in2, b, jnp.where(in1, c, d)))
    state = (
        sel3(jnp.int32(P3), jnp.int32(P2), jnp.int32(P1), jnp.int32(0)),
        jnp.minimum(
            sel3(jnp.int32(BITS_HI), jnp.int32(P3), jnp.int32(P2),
                 jnp.int32(P1)),
            hi_cap),
        sel3(jnp.int32(KSEL), KSEL - c_ge_p3, KSEL - c_ge_p2, KSEL - c_ge_p1),
        sel3(jnp.int32(3 * SEG), jnp.int32(2 * SEG), jnp.int32(SEG),
             jnp.int32(0)),
        sel3(q3, q2, q1, q0),
    )

    def dbl(i, st):
        st = _partition_pass(b0, b1, st, lanes)
        st = _partition_pass(b1, b0, st, lanes)
        return st

    state = lax.fori_loop(0, 16, dbl, state)
    thresh = state[0]

    def fin_body(i):
        xc = row_v[pl.ds(i * L, L)]
        bits = plsc.bitcast(xc, jnp.int32) & jnp.int32(0x7FFFFFFF)
        row_v[pl.ds(i * L, L)] = jnp.where(bits >= thresh, xc, jnp.float32(0.0))

    plsc.parallel_loop(0, NCHUNK, unroll=4)(fin_body)
    pltpu.sync_copy(row_v, out_hbm.at[wid])


@functools.cache
def _kwta():
    mesh = plsc.VectorSubcoreMesh(core_axis_name="c", subcore_axis_name="s",
                                  num_cores=NC, num_subcores=NS)
    return pl.kernel(
        _kwta_body,
        out_type=jax.ShapeDtypeStruct((B, N), jnp.float32),
        mesh=mesh,
        scratch_types=[
            pltpu.VMEM((N,), jnp.float32),
            pltpu.VMEM((BUF4LEN,), jnp.int32),
            pltpu.VMEM((BUF2LEN,), jnp.int32),
        ],
        compiler_params=pltpu.CompilerParams(needs_layout_passes=False),
    )


@jax.jit
def kernel(input_tensor):
    return _kwta()(input_tensor)


# speedup vs baseline: 6.3810x; 1.0064x over previous
import functools

import jax
import jax.numpy as jnp
from jax import lax
from jax.experimental import pallas as pl
from jax.experimental.pallas import tpu as pltpu
from jax.experimental.pallas import tpu_sc as plsc

B = 32
N = 8192
KSEL = 256
L = 16
NCHUNK = N // L
NC = 2
NS = 16

SEG = N + 32
BUFLEN = 2 * SEG
HI_BASE = 0
LO_BASE = SEG

BITS_HI = 0x7F800001

PA = 0x3FF33333
PB = 0x401CCCCD


def _partition_pass(src, dst, state, lanes):
    lo, hi, goal, off, n = state
    done = (hi - lo) <= 1
    small = jnp.logical_not(done) & (n <= L)
    active = jnp.logical_not(done) & jnp.logical_not(small)
    mid = lo + ((hi - lo) >> 1)
    nch = jnp.where(active, (n + L - 1) // L, 0)

    def body(i, carry):
        phi, plo = carry
        base = i * L
        bits = src[pl.ds(off + base, L)]
        valid = lanes < (n - base)
        mhi = valid & (bits >= mid)
        mlo = valid & (bits < mid)
        chi = jnp.sum(mhi.astype(jnp.int32))
        clo = jnp.sum(mlo.astype(jnp.int32))
        plsc.store_compressed(dst.at[pl.ds(HI_BASE + phi, L)], bits, mask=mhi)
        plsc.store_compressed(dst.at[pl.ds(LO_BASE + plo, L)], bits, mask=mlo)
        return (phi + chi, plo + clo)

    chi_t, clo_t = lax.fori_loop(0, nch, body, (jnp.int32(0), jnp.int32(0)))

    v = src[pl.ds(off, L)]
    sk = plsc.sort_key_val(v, v, mask=lanes < n, descending=True)[0]
    pick = jnp.sum(jnp.where(lanes == goal - 1, sk, jnp.int32(0)))

    take_hi = chi_t >= goal
    lo2 = jnp.where(done, lo,
                    jnp.where(small, pick, jnp.where(take_hi, mid, lo)))
    hi2 = jnp.where(done, hi,
                    jnp.where(small, pick + 1, jnp.where(take_hi, hi, mid)))
    keep = done | small
    goal2 = jnp.where(keep | take_hi, goal, goal - chi_t)
    off2 = jnp.where(keep, off, jnp.where(take_hi, HI_BASE, LO_BASE))
    n2 = jnp.where(keep, n, jnp.where(take_hi, chi_t, clo_t))
    return (lo2, hi2, goal2, off2, n2)


def _kwta_body(x_hbm, out_hbm, row_v, b0, b1):
    wid = lax.axis_index("s") * NC + lax.axis_index("c")
    pltpu.sync_copy(x_hbm.at[wid], row_v)

    lanes = lax.iota(jnp.int32, L)

    def p1_body(i, carry):
        qm, cv, mx = carry
        xc = row_v[pl.ds(i * L, L)]
        bits = plsc.bitcast(xc, jnp.int32) & jnp.int32(0x7FFFFFFF)
        geb = bits >= PB
        m_mid = (bits >= PA) & jnp.logical_not(geb)
        c_mid = jnp.sum(m_mid.astype(jnp.int32))
        plsc.store_compressed(b0.at[pl.ds(qm, L)], bits, mask=m_mid)
        return (qm + c_mid, cv + geb.astype(jnp.int32), jnp.maximum(mx, bits))

    zv = jnp.zeros((L,), jnp.int32)
    n_mid, cv, mxv = plsc.parallel_loop(
        0, NCHUNK, carry=(jnp.int32(0), zv, zv), unroll=2)(p1_body)
    c_hi = jnp.sum(cv)
    hi_cap = jnp.max(mxv) + 1

    in_band = (c_hi < KSEL) & (c_hi + n_mid >= KSEL)

    s_lo = jnp.where(c_hi >= KSEL, jnp.int32(PB), jnp.int32(0))
    s_hi = jnp.where(c_hi >= KSEL, jnp.int32(BITS_HI), jnp.int32(PA))

    def fb(_):
        def fbody(i, q):
            xc = row_v[pl.ds(i * L, L)]
            bits = plsc.bitcast(xc, jnp.int32) & jnp.int32(0x7FFFFFFF)
            m = (bits >= s_lo) & (bits < s_hi)
            c = jnp.sum(m.astype(jnp.int32))
            plsc.store_compressed(b0.at[pl.ds(SEG + q, L)], bits, mask=m)
            return q + c

        return lax.fori_loop(0, NCHUNK, fbody, jnp.int32(0))

    nfb = lax.cond(in_band, lambda _: jnp.int32(0), fb, 0)

    state = (
        jnp.where(in_band, jnp.int32(PA), s_lo),
        jnp.minimum(jnp.where(in_band, jnp.int32(PB), s_hi), hi_cap),
        jnp.where(c_hi >= KSEL, jnp.int32(KSEL),
                  jnp.where(in_band, KSEL - c_hi, KSEL - c_hi - n_mid)),
        jnp.where(in_band, jnp.int32(HI_BASE), jnp.int32(SEG)),
        jnp.where(in_band, n_mid, nfb),
    )

    def dbl(i, st):
        st = _partition_pass(b0, b1, st, lanes)
        st = _partition_pass(b1, b0, st, lanes)
        return st

    state = lax.fori_loop(0, 16, dbl, state)
    thresh = state[0]

    def fin_body(i):
        xc = row_v[pl.ds(i * L, L)]
        bits = plsc.bitcast(xc, jnp.int32) & jnp.int32(0x7FFFFFFF)
        row_v[pl.ds(i * L, L)] = jnp.where(bits >= thresh, xc, jnp.float32(0.0))

    plsc.parallel_loop(0, NCHUNK, unroll=4)(fin_body)
    pltpu.sync_copy(row_v, out_hbm.at[wid])


@functools.cache
def _kwta():
    mesh = plsc.VectorSubcoreMesh(core_axis_name="c", subcore_axis_name="s",
                                  num_cores=NC, num_subcores=NS)
    return pl.kernel(
        _kwta_body,
        out_type=jax.ShapeDtypeStruct((B, N), jnp.float32),
        mesh=mesh,
        scratch_types=[
            pltpu.VMEM((N,), jnp.float32),
            pltpu.VMEM((BUFLEN,), jnp.int32),
            pltpu.VMEM((BUFLEN,), jnp.int32),
        ],
        compiler_params=pltpu.CompilerParams(needs_layout_passes=False),
    )


@jax.jit
def kernel(input_tensor):
    return _kwta()(input_tensor)


# speedup vs baseline: 6.6089x; 1.0357x over previous
import functools

import jax
import jax.numpy as jnp
from jax import lax
from jax.experimental import pallas as pl
from jax.experimental.pallas import tpu as pltpu
from jax.experimental.pallas import tpu_sc as plsc

B = 32
N = 8192
KSEL = 256
L = 16
NCHUNK = N // L
NC = 2
NS = 16

SEG = N + 32
BUFLEN = 2 * SEG
HI_BASE = 0
LO_BASE = SEG

BITS_HI = 0x7F800001

PA = 0x3FF33333
PB = 0x401CCCCD


def _partition_pass(src, dst, state, lanes):
    lo, hi, goal, off, n = state
    done = (hi - lo) <= 1
    small = jnp.logical_not(done) & (n <= L)
    active = jnp.logical_not(done) & jnp.logical_not(small)
    mid = lo + ((hi - lo) >> 1)
    nch = jnp.where(active, (n + L - 1) // L, 0)

    def body(i, carry):
        phi, plo = carry
        base = i * L
        bits = src[pl.ds(off + base, L)]
        valid = lanes < (n - base)
        mhi = valid & (bits >= mid)
        mlo = valid & (bits < mid)
        chi = jnp.sum(mhi.astype(jnp.int32))
        clo = jnp.sum(mlo.astype(jnp.int32))
        plsc.store_compressed(dst.at[pl.ds(HI_BASE + phi, L)], bits, mask=mhi)
        plsc.store_compressed(dst.at[pl.ds(LO_BASE + plo, L)], bits, mask=mlo)
        return (phi + chi, plo + clo)

    chi_t, clo_t = lax.fori_loop(0, nch, body, (jnp.int32(0), jnp.int32(0)))

    v = src[pl.ds(off, L)]
    sk = plsc.sort_key_val(v, v, mask=lanes < n, descending=True)[0]
    pick = jnp.sum(jnp.where(lanes == goal - 1, sk, jnp.int32(0)))

    take_hi = chi_t >= goal
    lo2 = jnp.where(done, lo,
                    jnp.where(small, pick, jnp.where(take_hi, mid, lo)))
    hi2 = jnp.where(done, hi,
                    jnp.where(small, pick + 1, jnp.where(take_hi, hi, mid)))
    keep = done | small
    goal2 = jnp.where(keep | take_hi, goal, goal - chi_t)
    off2 = jnp.where(keep, off, jnp.where(take_hi, HI_BASE, LO_BASE))
    n2 = jnp.where(keep, n, jnp.where(take_hi, chi_t, clo_t))
    return (lo2, hi2, goal2, off2, n2)


def _kwta_body(x_hbm, out_hbm, row_v, b0, b1):
    wid = lax.axis_index("s") * NC + lax.axis_index("c")
    pltpu.sync_copy(x_hbm.at[wid], row_v)

    lanes = lax.iota(jnp.int32, L)

    def p1_body(i, carry):
        qm, cv, mx = carry
        xc = row_v[pl.ds(i * L, L)]
        bits = plsc.bitcast(xc, jnp.int32) & jnp.int32(0x7FFFFFFF)
        geb = bits >= PB
        m_mid = (bits >= PA) & jnp.logical_not(geb)
        c_mid = jnp.sum(m_mid.astype(jnp.int32))
        plsc.store_compressed(b0.at[pl.ds(qm, L)], bits, mask=m_mid)
        return (qm + c_mid, cv + geb.astype(jnp.int32), jnp.maximum(mx, bits))

    zv = jnp.zeros((L,), jnp.int32)
    n_mid, cv, mxv = plsc.parallel_loop(
        0, NCHUNK, carry=(jnp.int32(0), zv, zv), unroll=2)(p1_body)
    c_hi = jnp.sum(cv)
    hi_cap = jnp.max(mxv) + 1

    in_band = (c_hi < KSEL) & (c_hi + n_mid >= KSEL)

    s_lo = jnp.where(c_hi >= KSEL, jnp.int32(PB), jnp.int32(0))
    s_hi = jnp.where(c_hi >= KSEL, jnp.int32(BITS_HI), jnp.int32(PA))

    def fb(_):
        def fbody(i, q):
            xc = row_v[pl.ds(i * L, L)]
            bits = plsc.bitcast(xc, jnp.int32) & jnp.int32(0x7FFFFFFF)
            m = (bits >= s_lo) & (bits < s_hi)
            c = jnp.sum(m.astype(jnp.int32))
            plsc.store_compressed(b0.at[pl.ds(SEG + q, L)], bits, mask=m)
            return q + c

        return lax.fori_loop(0, NCHUNK, fbody, jnp.int32(0))

    nfb = lax.cond(in_band, lambda _: jnp.int32(0), fb, 0)

    state = (
        jnp.where(in_band, jnp.int32(PA), s_lo),
        jnp.minimum(jnp.where(in_band, jnp.int32(PB), s_hi), hi_cap),
        jnp.where(c_hi >= KSEL, jnp.int32(KSEL),
                  jnp.where(in_band, KSEL - c_hi, KSEL - c_hi - n_mid)),
        jnp.where(in_band, jnp.int32(HI_BASE), jnp.int32(SEG)),
        jnp.where(in_band, n_mid, nfb),
    )

    def dbl(st):
        st = _partition_pass(b0, b1, st, lanes)
        st = _partition_pass(b1, b0, st, lanes)
        return st

    state = lax.while_loop(lambda st: (st[1] - st[0]) > 1, dbl, state)
    thresh = state[0]

    def fin_body(i):
        xc = row_v[pl.ds(i * L, L)]
        bits = plsc.bitcast(xc, jnp.int32) & jnp.int32(0x7FFFFFFF)
        row_v[pl.ds(i * L, L)] = jnp.where(bits >= thresh, xc, jnp.float32(0.0))

    plsc.parallel_loop(0, NCHUNK, unroll=4)(fin_body)
    pltpu.sync_copy(row_v, out_hbm.at[wid])


@functools.cache
def _kwta():
    mesh = plsc.VectorSubcoreMesh(core_axis_name="c", subcore_axis_name="s",
                                  num_cores=NC, num_subcores=NS)
    return pl.kernel(
        _kwta_body,
        out_type=jax.ShapeDtypeStruct((B, N), jnp.float32),
        mesh=mesh,
        scratch_types=[
            pltpu.VMEM((N,), jnp.float32),
            pltpu.VMEM((BUFLEN,), jnp.int32),
            pltpu.VMEM((BUFLEN,), jnp.int32),
        ],
        compiler_params=pltpu.CompilerParams(needs_layout_passes=False),
    )


@jax.jit
def kernel(input_tensor):
    return _kwta()(input_tensor)
